# baseline (speedup 1.0000x reference)
import functools

import jax
import jax.numpy as jnp
from jax import lax
from jax.experimental import pallas as pl
from jax.experimental.pallas import tpu as pltpu
from jax.experimental.pallas import tpu_sc as plsc

_CHUNK = 128


@functools.lru_cache(maxsize=None)
def _make_gather2(B, D, NC, NW, n_chunks):
    b_per_w = n_chunks * _CHUNK
    mesh = plsc.VectorSubcoreMesh(core_axis_name="c", subcore_axis_name="s")

    @functools.partial(
        pl.kernel,
        mesh=mesh,
        out_type=(
            jax.ShapeDtypeStruct((B, D), jnp.float32),
            jax.ShapeDtypeStruct((B, D), jnp.float32),
        ),
        scratch_types=[
            pltpu.VMEM((n_chunks, _CHUNK), jnp.int32),
            pltpu.VMEM((n_chunks, _CHUNK), jnp.int32),
            pltpu.VMEM((b_per_w, D), jnp.float32),
            pltpu.VMEM((b_per_w, D), jnp.float32),
            pltpu.SemaphoreType.DMA,
            pltpu.SemaphoreType.DMA,
        ],
        compiler_params=pltpu.CompilerParams(use_tc_tiling_on_sc=False),
    )
    def gather2(u_tab, i_tab, u_idx, i_idx, u_out, i_out,
                u_idx_v, i_idx_v, u_rows_v, i_rows_v, u_sem, i_sem):
        wid = lax.axis_index("s") * NC + lax.axis_index("c")
        base = wid * b_per_w
        pltpu.sync_copy(u_idx.at[wid], u_idx_v)
        pltpu.sync_copy(i_idx.at[wid], i_idx_v)
        copies = []
        for j in range(n_chunks):
            dst = pl.ds(j * _CHUNK, _CHUNK)
            copies.append(
                pltpu.async_copy(u_tab.at[u_idx_v.at[j]], u_rows_v.at[dst], u_sem))
            copies.append(
                pltpu.async_copy(i_tab.at[i_idx_v.at[j]], i_rows_v.at[dst], i_sem))
        for c in copies:
            c.wait()
        pltpu.sync_copy(u_rows_v, u_out.at[pl.ds(base, b_per_w)])
        pltpu.sync_copy(i_rows_v, i_out.at[pl.ds(base, b_per_w)])

    return gather2


def kernel(user_table, item_table, user_idx, item_idx):
    (B,) = user_idx.shape
    D = user_table.shape[1]
    info = plsc.get_sparse_core_info()
    NC, NS = info.num_cores, info.num_subcores
    NW = NC * NS
    n_chunks = B // (NW * _CHUNK)
    u_idx = user_idx.reshape(NW, n_chunks, _CHUNK)
    i_idx = item_idx.reshape(NW, n_chunks, _CHUNK)
    fn = _make_gather2(B, D, NC, NW, n_chunks)
    return fn(user_table, item_table, u_idx, i_idx)
